# baseline (speedup 1.0000x reference)
import functools

import jax
import jax.numpy as jnp
from jax import lax
from jax.experimental import pallas as pl
from jax.experimental.pallas import tpu as pltpu
from jax.experimental.pallas import tpu_sc as plsc

_B = 2
_NIN = 50000
_NOUT = 12500
_K = 16
_C = 128
_M = _B * _NOUT
_KC = _K * _C

_NC, _NS = 2, 16
_NW = _NC * _NS
_CHUNK = 128
_NBUF = 6


def _pad_idx(idx_flat, nrows):
    n = idx_flat.shape[0]
    quantum = _NW * _CHUNK
    npad = (-n) % quantum
    if npad:
        pad = (jnp.arange(npad, dtype=jnp.int32) * 8) % nrows
        idx_flat = jnp.concatenate([idx_flat, pad])
    return idx_flat


def _sc_gather_multi(table, jobs):
    c = table.shape[-1]
    specs = [(idx.shape[0] // _CHUNK, idx.shape[0] // (_CHUNK * _NW))
             for idx in jobs]
    mesh = plsc.VectorSubcoreMesh(
        core_axis_name="c", subcore_axis_name="s",
        num_cores=_NC, num_subcores=_NS)
    nj = len(jobs)
    max_cpw = max(cpw for _, cpw in specs)

    @functools.partial(
        pl.kernel,
        out_type=tuple(
            jax.ShapeDtypeStruct((n * _CHUNK, c), table.dtype)
            for n, _ in specs),
        mesh=mesh,
        scratch_types=[
            pltpu.VMEM((max_cpw * _CHUNK,), jnp.int32),
        ] + [pltpu.VMEM((_CHUNK, c), table.dtype) for _ in range(_NBUF)]
          + [pltpu.SemaphoreType.DMA for _ in range(2 * _NBUF)],
    )
    def gk(table_hbm, *refs):
        idx_refs = refs[:nj]
        out_refs = refs[nj:2 * nj]
        idx_v = refs[2 * nj]
        bufs = refs[2 * nj + 1:2 * nj + 1 + _NBUF]
        gsems = refs[2 * nj + 1 + _NBUF:2 * nj + 1 + 2 * _NBUF]
        ssems = refs[2 * nj + 1 + 2 * _NBUF:]
        wid = lax.axis_index("s") * _NC + lax.axis_index("c")

        for (nchunks, cpw), idx_hbm, out_hbm in zip(
                specs, idx_refs, out_refs):
            base = wid * cpw
            pltpu.sync_copy(idx_hbm.at[pl.ds(base * _CHUNK, cpw * _CHUNK)],
                            idx_v.at[pl.ds(0, cpw * _CHUNK)])

            def start_gather(t, b):
                pltpu.async_copy(
                    table_hbm.at[idx_v.at[pl.ds(t * _CHUNK, _CHUNK)]],
                    bufs[b], gsems[b])

            def out_slice(t):
                return out_hbm.at[pl.ds((base + t) * _CHUNK, _CHUNK)]

            for b in range(min(_NBUF, cpw)):
                start_gather(b, b)

            def body(i, carry):
                t0 = i * _NBUF
                for b in range(_NBUF):
                    @pl.when(t0 + b < cpw)
                    def _():
                        pltpu.make_async_copy(
                            table_hbm.at[idx_v.at[pl.ds(0, _CHUNK)]],
                            bufs[b], gsems[b]).wait()
                        pltpu.async_copy(
                            bufs[b], out_slice(t0 + b), ssems[b])
                for b in range(_NBUF):
                    @pl.when(t0 + b < cpw)
                    def _():
                        pltpu.make_async_copy(
                            bufs[b], out_slice(t0 + b), ssems[b]).wait()

                    nxt = t0 + _NBUF + b

                    @pl.when(nxt < cpw)
                    def _():
                        start_gather(nxt, b)
                return carry

            lax.fori_loop(0, -(-cpw // _NBUF), body, 0)

    return gk(table, *jobs)


_BM = 1000
_GRID = _M // _BM
_SEG = _M // _BM


def _g_specs():
    return [pl.BlockSpec((_BM, _C), functools.partial(
        lambda k, i: (k * _SEG + i, 0), k)) for k in range(_K)]


def _tc_conv1(gm, d, w1r, wd, b1, bd):
    def body(*refs):
        g_refs = refs[:_K]
        d_ref, w1_ref, wd_ref, b1_ref, bd_ref, h1_ref, hd_ref, st_ref = \
            refs[_K:]
        w1 = w1_ref[...]
        h1 = b1_ref[...] + jnp.zeros((_BM, _C), jnp.float32)
        for k in range(_K):
            h1 = h1 + jnp.dot(g_refs[k][...],
                              w1[k * _C:(k + 1) * _C, :],
                              preferred_element_type=jnp.float32)
        hd = jnp.dot(d_ref[...], wd_ref[...],
                     preferred_element_type=jnp.float32) + bd_ref[...]
        h1_ref[...] = h1
        hd_ref[...] = hd
        st_ref[0, 0, :] = jnp.sum(h1, 0)
        st_ref[0, 1, :] = jnp.sum(h1 * h1, 0)
        st_ref[0, 2, :] = jnp.sum(hd, 0)
        st_ref[0, 3, :] = jnp.sum(hd * hd, 0)

    return pl.pallas_call(
        body,
        grid=(_GRID,),
        in_specs=_g_specs() + [
            pl.BlockSpec((_BM, _C), lambda i: (i, 0)),
            pl.BlockSpec((_KC, _C), lambda i: (0, 0)),
            pl.BlockSpec((_C, _C), lambda i: (0, 0)),
            pl.BlockSpec((1, _C), lambda i: (0, 0)),
            pl.BlockSpec((1, _C), lambda i: (0, 0)),
        ],
        out_specs=[
            pl.BlockSpec((_BM, _C), lambda i: (i, 0)),
            pl.BlockSpec((_BM, _C), lambda i: (i, 0)),
            pl.BlockSpec((1, 4, _C), lambda i: (i, 0, 0)),
        ],
        out_shape=[
            jax.ShapeDtypeStruct((_M, _C), jnp.float32),
            jax.ShapeDtypeStruct((_M, _C), jnp.float32),
            jax.ShapeDtypeStruct((_GRID, 4, _C), jnp.float32),
        ],
    )(*([gm] * _K), d, w1r, wd, b1, bd)


def _bn_scale_shift(st_blocks, j, gamma, beta, eps=1e-5):
    mean = jnp.sum(st_blocks[:, j, :], axis=0, keepdims=True) / _M
    var = jnp.sum(st_blocks[:, j + 1, :], axis=0, keepdims=True) / _M \
        - mean * mean
    scale = gamma / jnp.sqrt(var + eps)
    return scale, beta - mean * scale


def _tc_conv2(g2m, w2r, b2, st1, g1, be1):
    def body(*refs):
        g_refs = refs[:_K]
        w2_ref, b2_ref, st1_ref, g1_ref, be1_ref, h2_ref, st_ref = \
            refs[_K:]
        w2 = w2_ref[...]
        s1v, t1v = _bn_scale_shift(st1_ref[...], 0, g1_ref[...],
                                   be1_ref[...])
        h2 = b2_ref[...] + jnp.zeros((_BM, _C), jnp.float32)
        for k in range(_K):
            a = jnp.maximum(g_refs[k][...] * s1v + t1v, 0.0)
            h2 = h2 + jnp.dot(a,
                              w2[k * _C:(k + 1) * _C, :],
                              preferred_element_type=jnp.float32)
        h2_ref[...] = h2
        st_ref[0, 0, :] = jnp.sum(h2, 0)
        st_ref[0, 1, :] = jnp.sum(h2 * h2, 0)

    return pl.pallas_call(
        body,
        grid=(_GRID,),
        in_specs=_g_specs() + [
            pl.BlockSpec((_KC, _C), lambda i: (0, 0)),
            pl.BlockSpec((1, _C), lambda i: (0, 0)),
            pl.BlockSpec((_GRID, 4, _C), lambda i: (0, 0, 0)),
            pl.BlockSpec((1, _C), lambda i: (0, 0)),
            pl.BlockSpec((1, _C), lambda i: (0, 0)),
        ],
        out_specs=[
            pl.BlockSpec((_BM, _C), lambda i: (i, 0)),
            pl.BlockSpec((1, 2, _C), lambda i: (i, 0, 0)),
        ],
        out_shape=[
            jax.ShapeDtypeStruct((_M, _C), jnp.float32),
            jax.ShapeDtypeStruct((_GRID, 2, _C), jnp.float32),
        ],
    )(*([g2m] * _K), w2r, b2, st1, g1, be1)


def _tc_final(h2, hd, st1, st2, g2, be2, gd, bed):
    bm = 5000

    def body(h2_ref, hd_ref, st1_ref, st2_ref, g2_ref, be2_ref,
             gd_ref, bed_ref, o_ref):
        s2, t2 = _bn_scale_shift(st2_ref[...], 0, g2_ref[...], be2_ref[...])
        sd, td = _bn_scale_shift(st1_ref[...], 2, gd_ref[...], bed_ref[...])
        o_ref[...] = jnp.maximum(
            h2_ref[...] * s2 + t2 + hd_ref[...] * sd + td, 0.0)

    return pl.pallas_call(
        body,
        grid=(_M // bm,),
        in_specs=[
            pl.BlockSpec((bm, _C), lambda i: (i, 0)),
            pl.BlockSpec((bm, _C), lambda i: (i, 0)),
            pl.BlockSpec((_GRID, 4, _C), lambda i: (0, 0, 0)),
            pl.BlockSpec((_GRID, 2, _C), lambda i: (0, 0, 0)),
            pl.BlockSpec((1, _C), lambda i: (0, 0)),
            pl.BlockSpec((1, _C), lambda i: (0, 0)),
            pl.BlockSpec((1, _C), lambda i: (0, 0)),
            pl.BlockSpec((1, _C), lambda i: (0, 0)),
        ],
        out_specs=pl.BlockSpec((bm, _C), lambda i: (i, 0)),
        out_shape=jax.ShapeDtypeStruct((_M, _C), jnp.float32),
    )(h2, hd, st1, st2, g2, be2, gd, bed)


def kernel(x, knn1, knn2, ds_idx, W1, b1, W2, b2, Wd, bd,
           g1, be1, g2, be2, gd, bed):
    xf = x.reshape(_B * _NIN, _C)
    boff_in = (jnp.arange(_B, dtype=jnp.int32) * _NIN)[None, :, None]
    boff_out = (jnp.arange(_B, dtype=jnp.int32) * _NOUT)[None, :, None]
    idx1 = (knn1.T[:, None, :] + boff_in).reshape(-1)
    idx2 = (knn2.T[:, None, :] + boff_out).reshape(-1)
    idxd = (ds_idx[None, :] + boff_in[0]).reshape(-1)

    idx1 = _pad_idx(idx1, _B * _NIN)
    idxd = _pad_idx(idxd, _B * _NIN)
    idx2 = _pad_idx(idx2, _M)

    w1r = W1.reshape(_KC, _C)
    w2r = W2.reshape(_KC, _C)
    wd = Wd

    g1m, dsg = _sc_gather_multi(xf, [idx1, idxd])

    h1, hd, st1 = _tc_conv1(g1m, dsg, w1r, wd, b1[None], bd[None])

    (g2m,) = _sc_gather_multi(h1, [idx2])

    h2, st2 = _tc_conv2(g2m, w2r, b2[None], st1, g1[None], be1[None])

    out = _tc_final(h2, hd, st1, st2,
                    g2[None], be2[None], gd[None], bed[None])
    return out.reshape(_B, _NOUT, _C)


# speedup vs baseline: 1.0038x; 1.0038x over previous
import functools

import jax
import jax.numpy as jnp
from jax import lax
from jax.experimental import pallas as pl
from jax.experimental.pallas import tpu as pltpu
from jax.experimental.pallas import tpu_sc as plsc

_B = 2
_NIN = 50000
_NOUT = 12500
_K = 16
_C = 128
_M = _B * _NOUT
_KC = _K * _C

_NC, _NS = 2, 16
_NW = _NC * _NS
_CHUNK = 128
_NBUF = 7


def _pad_idx(idx_flat, nrows):
    n = idx_flat.shape[0]
    quantum = _NW * _CHUNK
    npad = (-n) % quantum
    if npad:
        pad = (jnp.arange(npad, dtype=jnp.int32) * 8) % nrows
        idx_flat = jnp.concatenate([idx_flat, pad])
    return idx_flat


def _sc_gather_multi(table, jobs):
    c = table.shape[-1]
    specs = [(idx.shape[0] // _CHUNK, idx.shape[0] // (_CHUNK * _NW))
             for idx in jobs]
    mesh = plsc.VectorSubcoreMesh(
        core_axis_name="c", subcore_axis_name="s",
        num_cores=_NC, num_subcores=_NS)
    nj = len(jobs)
    max_cpw = max(cpw for _, cpw in specs)

    @functools.partial(
        pl.kernel,
        out_type=tuple(
            jax.ShapeDtypeStruct((n * _CHUNK, c), table.dtype)
            for n, _ in specs),
        mesh=mesh,
        scratch_types=[
            pltpu.VMEM((max_cpw * _CHUNK,), jnp.int32),
        ] + [pltpu.VMEM((_CHUNK, c), table.dtype) for _ in range(_NBUF)]
          + [pltpu.SemaphoreType.DMA for _ in range(2 * _NBUF)],
    )
    def gk(table_hbm, *refs):
        idx_refs = refs[:nj]
        out_refs = refs[nj:2 * nj]
        idx_v = refs[2 * nj]
        bufs = refs[2 * nj + 1:2 * nj + 1 + _NBUF]
        gsems = refs[2 * nj + 1 + _NBUF:2 * nj + 1 + 2 * _NBUF]
        ssems = refs[2 * nj + 1 + 2 * _NBUF:]
        wid = lax.axis_index("s") * _NC + lax.axis_index("c")

        for (nchunks, cpw), idx_hbm, out_hbm in zip(
                specs, idx_refs, out_refs):
            base = wid * cpw
            pltpu.sync_copy(idx_hbm.at[pl.ds(base * _CHUNK, cpw * _CHUNK)],
                            idx_v.at[pl.ds(0, cpw * _CHUNK)])

            def start_gather(t, b):
                pltpu.async_copy(
                    table_hbm.at[idx_v.at[pl.ds(t * _CHUNK, _CHUNK)]],
                    bufs[b], gsems[b])

            def out_slice(t):
                return out_hbm.at[pl.ds((base + t) * _CHUNK, _CHUNK)]

            for b in range(min(_NBUF, cpw)):
                start_gather(b, b)

            def body(i, carry):
                t0 = i * _NBUF
                for b in range(_NBUF):
                    @pl.when(t0 + b < cpw)
                    def _():
                        pltpu.make_async_copy(
                            table_hbm.at[idx_v.at[pl.ds(0, _CHUNK)]],
                            bufs[b], gsems[b]).wait()
                        pltpu.async_copy(
                            bufs[b], out_slice(t0 + b), ssems[b])
                for b in range(_NBUF):
                    @pl.when(t0 + b < cpw)
                    def _():
                        pltpu.make_async_copy(
                            bufs[b], out_slice(t0 + b), ssems[b]).wait()

                    nxt = t0 + _NBUF + b

                    @pl.when(nxt < cpw)
                    def _():
                        start_gather(nxt, b)
                return carry

            lax.fori_loop(0, -(-cpw // _NBUF), body, 0)

    return gk(table, *jobs)


_BM = 1000
_GRID = _M // _BM
_SEG = _M // _BM


def _g_specs():
    return [pl.BlockSpec((_BM, _C), functools.partial(
        lambda k, i: (k * _SEG + i, 0), k)) for k in range(_K)]


def _tc_conv1(gm, d, w1r, wd, b1, bd):
    def body(*refs):
        g_refs = refs[:_K]
        d_ref, w1_ref, wd_ref, b1_ref, bd_ref, h1_ref, hd_ref, st_ref = \
            refs[_K:]
        w1 = w1_ref[...]
        h1 = b1_ref[...] + jnp.zeros((_BM, _C), jnp.float32)
        for k in range(_K):
            h1 = h1 + jnp.dot(g_refs[k][...],
                              w1[k * _C:(k + 1) * _C, :],
                              preferred_element_type=jnp.float32)
        hd = jnp.dot(d_ref[...], wd_ref[...],
                     preferred_element_type=jnp.float32) + bd_ref[...]
        h1_ref[...] = h1
        hd_ref[...] = hd
        st_ref[0, 0, :] = jnp.sum(h1, 0)
        st_ref[0, 1, :] = jnp.sum(h1 * h1, 0)
        st_ref[0, 2, :] = jnp.sum(hd, 0)
        st_ref[0, 3, :] = jnp.sum(hd * hd, 0)

    return pl.pallas_call(
        body,
        grid=(_GRID,),
        in_specs=_g_specs() + [
            pl.BlockSpec((_BM, _C), lambda i: (i, 0)),
            pl.BlockSpec((_KC, _C), lambda i: (0, 0)),
            pl.BlockSpec((_C, _C), lambda i: (0, 0)),
            pl.BlockSpec((1, _C), lambda i: (0, 0)),
            pl.BlockSpec((1, _C), lambda i: (0, 0)),
        ],
        out_specs=[
            pl.BlockSpec((_BM, _C), lambda i: (i, 0)),
            pl.BlockSpec((_BM, _C), lambda i: (i, 0)),
            pl.BlockSpec((1, 4, _C), lambda i: (i, 0, 0)),
        ],
        out_shape=[
            jax.ShapeDtypeStruct((_M, _C), jnp.float32),
            jax.ShapeDtypeStruct((_M, _C), jnp.float32),
            jax.ShapeDtypeStruct((_GRID, 4, _C), jnp.float32),
        ],
    )(*([gm] * _K), d, w1r, wd, b1, bd)


def _bn_scale_shift(st_blocks, j, gamma, beta, eps=1e-5):
    mean = jnp.sum(st_blocks[:, j, :], axis=0, keepdims=True) / _M
    var = jnp.sum(st_blocks[:, j + 1, :], axis=0, keepdims=True) / _M \
        - mean * mean
    scale = gamma / jnp.sqrt(var + eps)
    return scale, beta - mean * scale


def _tc_conv2(g2m, w2r, b2, st1, g1, be1):
    def body(*refs):
        g_refs = refs[:_K]
        w2_ref, b2_ref, st1_ref, g1_ref, be1_ref, h2_ref, st_ref = \
            refs[_K:]
        w2 = w2_ref[...]
        s1v, t1v = _bn_scale_shift(st1_ref[...], 0, g1_ref[...],
                                   be1_ref[...])
        h2 = b2_ref[...] + jnp.zeros((_BM, _C), jnp.float32)
        for k in range(_K):
            a = jnp.maximum(g_refs[k][...] * s1v + t1v, 0.0)
            h2 = h2 + jnp.dot(a,
                              w2[k * _C:(k + 1) * _C, :],
                              preferred_element_type=jnp.float32)
        h2_ref[...] = h2
        st_ref[0, 0, :] = jnp.sum(h2, 0)
        st_ref[0, 1, :] = jnp.sum(h2 * h2, 0)

    return pl.pallas_call(
        body,
        grid=(_GRID,),
        in_specs=_g_specs() + [
            pl.BlockSpec((_KC, _C), lambda i: (0, 0)),
            pl.BlockSpec((1, _C), lambda i: (0, 0)),
            pl.BlockSpec((_GRID, 4, _C), lambda i: (0, 0, 0)),
            pl.BlockSpec((1, _C), lambda i: (0, 0)),
            pl.BlockSpec((1, _C), lambda i: (0, 0)),
        ],
        out_specs=[
            pl.BlockSpec((_BM, _C), lambda i: (i, 0)),
            pl.BlockSpec((1, 2, _C), lambda i: (i, 0, 0)),
        ],
        out_shape=[
            jax.ShapeDtypeStruct((_M, _C), jnp.float32),
            jax.ShapeDtypeStruct((_GRID, 2, _C), jnp.float32),
        ],
    )(*([g2m] * _K), w2r, b2, st1, g1, be1)


def _tc_final(h2, hd, st1, st2, g2, be2, gd, bed):
    bm = 5000

    def body(h2_ref, hd_ref, st1_ref, st2_ref, g2_ref, be2_ref,
             gd_ref, bed_ref, o_ref):
        s2, t2 = _bn_scale_shift(st2_ref[...], 0, g2_ref[...], be2_ref[...])
        sd, td = _bn_scale_shift(st1_ref[...], 2, gd_ref[...], bed_ref[...])
        o_ref[...] = jnp.maximum(
            h2_ref[...] * s2 + t2 + hd_ref[...] * sd + td, 0.0)

    return pl.pallas_call(
        body,
        grid=(_M // bm,),
        in_specs=[
            pl.BlockSpec((bm, _C), lambda i: (i, 0)),
            pl.BlockSpec((bm, _C), lambda i: (i, 0)),
            pl.BlockSpec((_GRID, 4, _C), lambda i: (0, 0, 0)),
            pl.BlockSpec((_GRID, 2, _C), lambda i: (0, 0, 0)),
            pl.BlockSpec((1, _C), lambda i: (0, 0)),
            pl.BlockSpec((1, _C), lambda i: (0, 0)),
            pl.BlockSpec((1, _C), lambda i: (0, 0)),
            pl.BlockSpec((1, _C), lambda i: (0, 0)),
        ],
        out_specs=pl.BlockSpec((bm, _C), lambda i: (i, 0)),
        out_shape=jax.ShapeDtypeStruct((_M, _C), jnp.float32),
    )(h2, hd, st1, st2, g2, be2, gd, bed)


def kernel(x, knn1, knn2, ds_idx, W1, b1, W2, b2, Wd, bd,
           g1, be1, g2, be2, gd, bed):
    xf = x.reshape(_B * _NIN, _C)
    boff_in = (jnp.arange(_B, dtype=jnp.int32) * _NIN)[None, :, None]
    boff_out = (jnp.arange(_B, dtype=jnp.int32) * _NOUT)[None, :, None]
    idx1 = (knn1.T[:, None, :] + boff_in).reshape(-1)
    idx2 = (knn2.T[:, None, :] + boff_out).reshape(-1)
    idxd = (ds_idx[None, :] + boff_in[0]).reshape(-1)

    idx1 = _pad_idx(idx1, _B * _NIN)
    idxd = _pad_idx(idxd, _B * _NIN)
    idx2 = _pad_idx(idx2, _M)

    w1r = W1.reshape(_KC, _C)
    w2r = W2.reshape(_KC, _C)
    wd = Wd

    g1m, dsg = _sc_gather_multi(xf, [idx1, idxd])

    h1, hd, st1 = _tc_conv1(g1m, dsg, w1r, wd, b1[None], bd[None])

    (g2m,) = _sc_gather_multi(h1, [idx2])

    h2, st2 = _tc_conv2(g2m, w2r, b2[None], st1, g1[None], be1[None])

    out = _tc_final(h2, hd, st1, st2,
                    g2[None], be2[None], gd[None], bed[None])
    return out.reshape(_B, _NOUT, _C)
